# baseline (speedup 1.0000x reference)
import functools
import math

import jax
import jax.numpy as jnp
from jax import lax
from jax.experimental import pallas as pl
from jax.experimental.pallas import tpu as pltpu
from jax.experimental.pallas import tpu_sc as plsc

N = 10000
E = 160000
IN = 256
H = 2
C = 256
NEG_SLOPE = 0.2


def _a_kernel(x_ref, w_ref, asrc_ref, adst_ref, h_out_ref, a_out_ref, m_out_ref):
    i = pl.program_id(0)
    x = x_ref[...]
    h = jnp.dot(x, w_ref[...], preferred_element_type=jnp.float32)
    h_out_ref[...] = h
    att_s = asrc_ref[...]
    att_d = adst_ref[...]
    a_s0 = jnp.dot(h[:, :C], att_s[0], precision=lax.Precision.HIGHEST)
    a_s1 = jnp.dot(h[:, C:], att_s[1], precision=lax.Precision.HIGHEST)
    a_d0 = jnp.dot(h[:, :C], att_d[0], precision=lax.Precision.HIGHEST)
    a_d1 = jnp.dot(h[:, C:], att_d[1], precision=lax.Precision.HIGHEST)
    blk = jnp.stack([a_s0, a_s1, a_d0, a_d1, a_s0, a_s1, a_d0, a_d1], axis=1)
    a_out_ref[...] = blk
    bm = jnp.max(blk, axis=0)[None, :]
    bm = jnp.broadcast_to(bm.T, (8, 128))

    @pl.when(i == 0)
    def _():
        m_out_ref[...] = bm

    @pl.when(i != 0)
    def _():
        m_out_ref[...] = jnp.maximum(m_out_ref[...], bm)


def _run_a(x, W, att_src, att_dst):
    bm = 2000
    grid = (N // bm,)
    h_all, a_all, m_all = pl.pallas_call(
        _a_kernel,
        grid=grid,
        in_specs=[
            pl.BlockSpec((bm, IN), lambda i: (i, 0)),
            pl.BlockSpec((IN, H * C), lambda i: (0, 0)),
            pl.BlockSpec((H, C), lambda i: (0, 0)),
            pl.BlockSpec((H, C), lambda i: (0, 0)),
        ],
        out_specs=[
            pl.BlockSpec((bm, H * C), lambda i: (i, 0)),
            pl.BlockSpec((bm, 8), lambda i: (i, 0)),
            pl.BlockSpec((8, 128), lambda i: (0, 0)),
        ],
        out_shape=[
            jax.ShapeDtypeStruct((N, H * C), jnp.float32),
            jax.ShapeDtypeStruct((N, 8), jnp.float32),
            jax.ShapeDtypeStruct((8, 128), jnp.float32),
        ],
    )(x, W, att_src.reshape(H, C), att_dst.reshape(H, C))
    return h_all, a_all, m_all


def _c_kernel(z_ref, s_ref, a_ref, m_ref, h_ref, bg_ref,
              wa_ref, ba_ref, w1_ref, b1_ref, w2_ref, b2_ref,
              w3_ref, b3_ref, w4_ref, b4_ref, out_ref):
    z = z_ref[...]
    s_e = s_ref[...]
    a = a_ref[...]
    m0 = m_ref[0, 0]
    m1 = m_ref[1, 0]
    hh = h_ref[...]
    t0 = a[:, 0] + a[:, 2]
    t1 = a[:, 1] + a[:, 3]
    t0 = jnp.where(t0 > 0, t0, NEG_SLOPE * t0)
    t1 = jnp.where(t1 > 0, t1, NEG_SLOPE * t1)
    w_self0 = jnp.exp(t0 - m0)
    w_self1 = jnp.exp(t1 - m1)
    s0 = s_e[:, 0] + w_self0 + 1e-16
    s1 = s_e[:, 1] + w_self1 + 1e-16
    zz0 = (z[:, :C] + w_self0[:, None] * hh[:, :C]) / s0[:, None]
    zz1 = (z[:, C:] + w_self1[:, None] * hh[:, C:]) / s1[:, None]
    gat = jnp.concatenate([zz0, zz1], axis=1) + bg_ref[...][None, :]
    gat = jnp.maximum(gat, 0.0)
    t = jnp.maximum(jnp.dot(gat, wa_ref[...], preferred_element_type=jnp.float32) + ba_ref[...][None, :], 0.0)
    t = jnp.maximum(jnp.dot(t, w1_ref[...], preferred_element_type=jnp.float32) + b1_ref[...][None, :], 0.0)
    t = jnp.maximum(jnp.dot(t, w2_ref[...], preferred_element_type=jnp.float32) + b2_ref[...][None, :], 0.0)
    t = jnp.maximum(jnp.dot(t, w3_ref[...], preferred_element_type=jnp.float32) + b3_ref[...][None, :], 0.0)
    emb = jnp.dot(t, w4_ref[...], preferred_element_type=jnp.float32) + b4_ref[...][None, :]
    out_ref[...] = jnp.concatenate(
        [emb, jnp.zeros((emb.shape[0], 128 - emb.shape[1]), jnp.float32)], axis=1)


def _run_c(z, s, a_all, m_all, h_all, b_gat, Wa, ba, W1, b1, W2, b2, W3, b3, W4, b4):
    bm = 2000
    grid = (N // bm,)
    return pl.pallas_call(
        _c_kernel,
        grid=grid,
        in_specs=[
            pl.BlockSpec((bm, H * C), lambda i: (i, 0)),
            pl.BlockSpec((bm, 2), lambda i: (i, 0)),
            pl.BlockSpec((bm, 8), lambda i: (i, 0)),
            pl.BlockSpec((8, 128), lambda i: (0, 0)),
            pl.BlockSpec((bm, H * C), lambda i: (i, 0)),
            pl.BlockSpec((H * C,), lambda i: (0,)),
            pl.BlockSpec((512, 256), lambda i: (0, 0)),
            pl.BlockSpec((256,), lambda i: (0,)),
            pl.BlockSpec((256, 128), lambda i: (0, 0)),
            pl.BlockSpec((128,), lambda i: (0,)),
            pl.BlockSpec((128, 64), lambda i: (0, 0)),
            pl.BlockSpec((64,), lambda i: (0,)),
            pl.BlockSpec((64, 32), lambda i: (0, 0)),
            pl.BlockSpec((32,), lambda i: (0,)),
            pl.BlockSpec((32, 3), lambda i: (0, 0)),
            pl.BlockSpec((3,), lambda i: (0,)),
        ],
        out_specs=pl.BlockSpec((bm, 128), lambda i: (i, 0)),
        out_shape=jax.ShapeDtypeStruct((N, 128), jnp.float32),
    )(z, s, a_all, m_all, h_all, b_gat, Wa, ba, W1, b1, W2, b2, W3, b3, W4, b4)


def _d_kernel(ei_ref, ej_ref, out_ref):
    oi = ei_ref[...]
    oj = ej_ref[...]
    sqi = jnp.sum(oi * oi, axis=1)
    sqj = jnp.sum(oj * oj, axis=1)
    d2 = sqi[:, None] + sqj[None, :] - 2.0 * jnp.dot(
        oi, oj.T, preferred_element_type=jnp.float32)
    d2 = jnp.maximum(d2, 0.0)
    mask = d2 > 1e-12
    out_ref[...] = jnp.where(mask, jnp.sqrt(jnp.where(mask, d2, 1.0)), 0.0)


def _run_d(embp):
    bi = 200
    grid = (N // bi,)
    return pl.pallas_call(
        _d_kernel,
        grid=grid,
        in_specs=[
            pl.BlockSpec((bi, 128), lambda i: (i, 0)),
            pl.BlockSpec((N, 128), lambda i: (0, 0)),
        ],
        out_specs=pl.BlockSpec((bi, N), lambda i: (i, 0)),
        out_shape=jax.ShapeDtypeStruct((N, N), jnp.float32),
    )(embp, embp)


HALF = N // 2
ROUND = 1256
NR = 4
EPT = E // 16
EBLK = 2000
CAPQ = 1552
ZL = 1280


def _splat(v, k):
    return v[jnp.full((16,), k, jnp.int32)]


def _sc_body(src_hbm, dst_hbm, a4_hbm, m_hbm, h_hbm, z_hbm, s_hbm, sscr_hbm,
             a4_v, m_v, se_v, de_v, s_part,
             bsrc, bdst,
             rows0, rows1, sidx0, didx0, sidx1, didx1,
             row_tmp, acc_v, cnts_s,
             z_sp, sem0, sem0b, sem1):
    cid = lax.axis_index("c")
    sid = lax.axis_index("s")
    zeros16 = jnp.zeros((16,), jnp.float32)
    iota16 = lax.iota(jnp.int32, 16)

    pltpu.sync_copy(a4_hbm, a4_v)
    pltpu.sync_copy(m_hbm, m_v)
    mv = m_v[...]
    m0s = _splat(mv, 0)
    m1s = _splat(mv, 1)
    half_lo = cid * HALF

    def _zs(i, _):
        s_part[pl.ds(i * 16, 16)] = zeros16
        return 0
    lax.fori_loop(0, N // 16, _zs, 0)

    def _edge_w(srcv, dstv):
        as0 = plsc.load_gather(a4_v, [srcv])
        as1 = plsc.load_gather(a4_v, [N + srcv])
        ad0 = plsc.load_gather(a4_v, [2 * N + dstv])
        ad1 = plsc.load_gather(a4_v, [3 * N + dstv])
        e0 = as0 + ad0
        e1 = as1 + ad1
        e0 = jnp.where(e0 > 0, e0, NEG_SLOPE * e0)
        e1 = jnp.where(e1 > 0, e1, NEG_SLOPE * e1)
        return jnp.exp(e0 - m0s), jnp.exp(e1 - m1s)

    ebase = sid * EPT
    carry0 = (jnp.int32(0),) * NR

    def _blk(b, carry):
        pltpu.sync_copy(src_hbm.at[pl.ds(ebase + b * EBLK, EBLK)], se_v)
        pltpu.sync_copy(dst_hbm.at[pl.ds(ebase + b * EBLK, EBLK)], de_v)

        def _p1(g, pc):
            srcv = se_v[pl.ds(g * 16, 16)]
            dstv = de_v[pl.ds(g * 16, 16)]
            w0, w1 = _edge_w(srcv, dstv)
            dl = dstv - half_lo
            own = (dl >= 0) & (dl < HALF)
            plsc.addupdate_scatter(s_part, [2 * dl], w0, mask=own)
            plsc.addupdate_scatter(s_part, [2 * dl + 1], w1, mask=own)
            q = dl // ROUND
            dq = dl - q * ROUND
            newpc = []
            for rr in range(NR):
                mk = own & (q == rr)
                o = jnp.minimum(pc[rr], CAPQ - 16)
                plsc.store_compressed(bsrc.at[pl.ds(rr * CAPQ + o, 16)], srcv, mask=mk)
                plsc.store_compressed(bdst.at[pl.ds(rr * CAPQ + o, 16)], dq, mask=mk)
                newpc.append(pc[rr] + jnp.sum(mk.astype(jnp.int32)))
            return tuple(newpc)

        return lax.fori_loop(0, EBLK // 16, _p1, carry)

    cnts = lax.fori_loop(0, EPT // EBLK, _blk, carry0)
    for rr in range(NR):
        cnts_s[rr] = jnp.minimum(cnts[rr], CAPQ - 16)

    myrow = (cid * 16 + sid) * N
    pltpu.sync_copy(s_part, sscr_hbm.at[pl.ds(myrow, N)])
    plsc.subcore_barrier()
    rbase = jnp.minimum(sid * 632, N - 640)

    def _zacc(i, _):
        acc_v[pl.ds(i * 16, 16)] = zeros16
        return 0
    lax.fori_loop(0, 40, _zacc, 0)

    def _sred(j, _):
        pltpu.sync_copy(sscr_hbm.at[pl.ds((cid * 16 + j) * N + rbase, 640)], row_tmp)
        def _add(k, __):
            acc_v[pl.ds(k * 16, 16)] = acc_v[pl.ds(k * 16, 16)] + row_tmp[pl.ds(k * 16, 16)]
            return 0
        lax.fori_loop(0, 40, _add, 0)
        return 0
    lax.fori_loop(0, 16, _sred, 0)
    pltpu.sync_copy(acc_v, s_hbm.at[pl.ds(cid * N + rbase, 640)])

    ii4 = iota16 // 4
    im4 = iota16 % 4

    def _zero_rows(rv):
        for jj in range(16):
            for t in range(4):
                for u in range(8):
                    rv[jj, t, pl.ds(u * 16, 16)] = zeros16

    def _build(c, qbase, cnt, si, di):
        lanes = (iota16 + c * 16) < cnt
        rb = bb + c * 16
        srcv = jnp.where(lanes, bsrc[pl.ds(rb, 16)], 0)
        dqv = jnp.where(lanes, bdst[pl.ds(rb, 16)], 0)
        si[...] = srcv
        di[...] = dqv
        w0, w1 = _edge_w(srcv, qbase + dqv)
        w0 = jnp.where(lanes, w0, 0.0)
        w1 = jnp.where(lanes, w1, 0.0)
        return w0, w1

    def _scale(rv, w0, w1):
        for k in range(16):
            s0 = _splat(w0, k)
            s1 = _splat(w1, k)
            for t in range(4):
                f = s0 if t < 2 else s1
                for u in range(8):
                    rv[k, t, pl.ds(u * 16, 16)] = rv[k, t, pl.ds(u * 16, 16)] * f

    def _round(r, _):
        qbase = half_lo + r * ROUND
        _zero_rows(rows0)
        zb4 = sid * 80
        for kz in range(5):
            pltpu.sync_copy(rows0, z_sp.at[pl.ds(zb4 + kz * 16, 16)])
        plsc.subcore_barrier()

        cnt = cnts_s[r]
        nch2 = (cnt + 31) // 32
        global bb
        bb = r * CAPQ

        w00, w10 = _build(0, qbase, cnt, sidx0, didx0)
        pltpu.async_copy(h_hbm.at[sidx0], rows0, sem0)
        w01, w11 = _build(1, qbase, cnt, sidx1, didx1)
        pltpu.async_copy(h_hbm.at[sidx1], rows1, sem0b)

        def _p2(j2, carry):
            wa0, wa1, wb0, wb1 = carry
            ca = 2 * j2
            pltpu.make_async_copy(h_hbm.at[sidx0], rows0, sem0).wait()
            _scale(rows0, wa0, wa1)
            pltpu.async_copy(rows0, z_sp.at[didx0], sem1, add=True)
            pltpu.make_async_copy(h_hbm.at[sidx1], rows1, sem0b).wait()
            _scale(rows1, wb0, wb1)
            pltpu.async_copy(rows1, z_sp.at[didx1], sem1, add=True)
            pltpu.make_async_copy(rows0, z_sp.at[didx0], sem1).wait()
            na0, na1 = _build(ca + 2, qbase, cnt, sidx0, didx0)
            pltpu.async_copy(h_hbm.at[sidx0], rows0, sem0)
            pltpu.make_async_copy(rows1, z_sp.at[didx1], sem1).wait()
            nb0, nb1 = _build(ca + 3, qbase, cnt, sidx1, didx1)
            pltpu.async_copy(h_hbm.at[sidx1], rows1, sem0b)
            return na0, na1, nb0, nb1

        lax.fori_loop(0, nch2, _p2, (w00, w10, w01, w11))
        pltpu.make_async_copy(h_hbm.at[sidx0], rows0, sem0).wait()
        pltpu.make_async_copy(h_hbm.at[sidx1], rows1, sem0b).wait()
        plsc.subcore_barrier()
        zmax = jnp.where(r == NR - 1, 1152, ROUND - 80)
        zst = jnp.minimum(sid * 80, zmax)
        pltpu.sync_copy(z_sp.at[pl.ds(zst, 80)],
                        z_hbm.at[pl.ds(qbase + zst, 80)])
        plsc.subcore_barrier()
        return 0

    lax.fori_loop(0, NR, _round, 0)


def _run_b(src, dst, a4, m16, h_all):
    mesh = plsc.VectorSubcoreMesh(core_axis_name="c", subcore_axis_name="s")
    f32 = jnp.float32
    i32 = jnp.int32
    sc = pl.kernel(
        _sc_body,
        mesh=mesh,
        compiler_params=pltpu.CompilerParams(needs_layout_passes=False),
        out_type=[
            jax.ShapeDtypeStruct((N, 4, 128), f32),
            jax.ShapeDtypeStruct((2 * N,), f32),
            jax.ShapeDtypeStruct((32 * N,), f32),
        ],
        scratch_types=[
            pltpu.VMEM((4 * N,), f32),
            pltpu.VMEM((16,), f32),
            pltpu.VMEM((EBLK,), i32),
            pltpu.VMEM((EBLK,), i32),
            pltpu.VMEM((N,), f32),
            pltpu.VMEM((NR * CAPQ,), i32),
            pltpu.VMEM((NR * CAPQ,), i32),
            pltpu.VMEM((16, 4, 128), f32),
            pltpu.VMEM((16, 4, 128), f32),
            pltpu.VMEM((16,), i32),
            pltpu.VMEM((16,), i32),
            pltpu.VMEM((16,), i32),
            pltpu.VMEM((16,), i32),
            pltpu.VMEM((640,), f32),
            pltpu.VMEM((640,), f32),
            pltpu.SMEM((8,), i32),
            pltpu.VMEM_SHARED((ZL, 4, 128), f32),
            pltpu.SemaphoreType.DMA,
            pltpu.SemaphoreType.DMA,
            pltpu.SemaphoreType.DMA,
        ],
    )
    h4 = h_all.reshape((N, 4, 128))
    z4, s_flat, _sscr = sc(src, dst, a4.reshape(-1), m16, h4)
    z = z4.reshape(N, H * C)
    return z, s_flat.reshape(N, 2)


def _edge_phase_xla(src, dst, a_all, m0, m1, h_all):
    e0 = a_all[:, 0][src] + a_all[:, 2][dst]
    e1 = a_all[:, 1][src] + a_all[:, 3][dst]
    e0 = jnp.where(e0 > 0, e0, NEG_SLOPE * e0)
    e1 = jnp.where(e1 > 0, e1, NEG_SLOPE * e1)
    w0 = jnp.exp(e0 - m0)
    w1 = jnp.exp(e1 - m1)
    hs = h_all[src]
    z0 = jax.ops.segment_sum(w0[:, None] * hs[:, :C], dst, num_segments=N)
    z1 = jax.ops.segment_sum(w1[:, None] * hs[:, C:], dst, num_segments=N)
    z = jnp.concatenate([z0, z1], axis=1)
    s0 = jax.ops.segment_sum(w0, dst, num_segments=N)
    s1 = jax.ops.segment_sum(w1, dst, num_segments=N)
    s = jnp.stack([s0, s1], axis=1)
    return z, s


def kernel(x, edge_index, W, att_src, att_dst, b_gat, Wa, ba, W1, b1,
           W2, b2, W3, b3, W4, b4):
    src = edge_index[0]
    dst = edge_index[1]
    h_all, a_all, m_all = _run_a(x, W, att_src, att_dst)
    t0 = m_all[0, 0] + m_all[2, 0]
    t1 = m_all[1, 0] + m_all[3, 0]
    m0 = jnp.where(t0 > 0, t0, NEG_SLOPE * t0)
    m1 = jnp.where(t1 > 0, t1, NEG_SLOPE * t1)
    m2 = jnp.zeros((8, 128), jnp.float32)
    m2 = m2.at[0, :].set(m0).at[1, :].set(m1)
    a4 = jnp.transpose(a_all)[:4]
    m16 = jnp.zeros((16,), jnp.float32).at[0].set(m0).at[1].set(m1)
    z, s = _run_b(src, dst, a4, m16, h_all)
    embp = _run_c(z, s, a_all, m2, h_all, b_gat, Wa, ba,
                  W1, b1, W2, b2, W3, b3, W4, b4)
    return _run_d(embp)


# speedup vs baseline: 1.0269x; 1.0269x over previous
import functools
import math

import jax
import jax.numpy as jnp
from jax import lax
from jax.experimental import pallas as pl
from jax.experimental.pallas import tpu as pltpu
from jax.experimental.pallas import tpu_sc as plsc

N = 10000
E = 160000
IN = 256
H = 2
C = 256
NEG_SLOPE = 0.2


def _a_kernel(x_ref, w_ref, asrc_ref, adst_ref, h_out_ref, a_out_ref, m_out_ref):
    i = pl.program_id(0)
    x = x_ref[...]
    h = jnp.dot(x, w_ref[...], preferred_element_type=jnp.float32)
    h_out_ref[...] = h
    att_s = asrc_ref[...]
    att_d = adst_ref[...]
    a_s0 = jnp.dot(h[:, :C], att_s[0], precision=lax.Precision.HIGHEST)
    a_s1 = jnp.dot(h[:, C:], att_s[1], precision=lax.Precision.HIGHEST)
    a_d0 = jnp.dot(h[:, :C], att_d[0], precision=lax.Precision.HIGHEST)
    a_d1 = jnp.dot(h[:, C:], att_d[1], precision=lax.Precision.HIGHEST)
    blk = jnp.stack([a_s0, a_s1, a_d0, a_d1, a_s0, a_s1, a_d0, a_d1], axis=1)
    a_out_ref[...] = blk
    bm = jnp.max(blk, axis=0)[None, :]
    bm = jnp.broadcast_to(bm.T, (8, 128))

    @pl.when(i == 0)
    def _():
        m_out_ref[...] = bm

    @pl.when(i != 0)
    def _():
        m_out_ref[...] = jnp.maximum(m_out_ref[...], bm)


def _run_a(x, W, att_src, att_dst):
    bm = 2000
    grid = (N // bm,)
    h_all, a_all, m_all = pl.pallas_call(
        _a_kernel,
        grid=grid,
        in_specs=[
            pl.BlockSpec((bm, IN), lambda i: (i, 0)),
            pl.BlockSpec((IN, H * C), lambda i: (0, 0)),
            pl.BlockSpec((H, C), lambda i: (0, 0)),
            pl.BlockSpec((H, C), lambda i: (0, 0)),
        ],
        out_specs=[
            pl.BlockSpec((bm, H * C), lambda i: (i, 0)),
            pl.BlockSpec((bm, 8), lambda i: (i, 0)),
            pl.BlockSpec((8, 128), lambda i: (0, 0)),
        ],
        out_shape=[
            jax.ShapeDtypeStruct((N, H * C), jnp.float32),
            jax.ShapeDtypeStruct((N, 8), jnp.float32),
            jax.ShapeDtypeStruct((8, 128), jnp.float32),
        ],
    )(x, W, att_src.reshape(H, C), att_dst.reshape(H, C))
    return h_all, a_all, m_all


def _c_kernel(z_ref, s_ref, a_ref, m_ref, h_ref, bg_ref,
              wa_ref, ba_ref, w1_ref, b1_ref, w2_ref, b2_ref,
              w3_ref, b3_ref, w4_ref, b4_ref, out_ref):
    z = z_ref[...]
    s_e = s_ref[...]
    a = a_ref[...]
    m0 = m_ref[0, 0]
    m1 = m_ref[1, 0]
    hh = h_ref[...]
    t0 = a[:, 0] + a[:, 2]
    t1 = a[:, 1] + a[:, 3]
    t0 = jnp.where(t0 > 0, t0, NEG_SLOPE * t0)
    t1 = jnp.where(t1 > 0, t1, NEG_SLOPE * t1)
    w_self0 = jnp.exp(t0 - m0)
    w_self1 = jnp.exp(t1 - m1)
    s0 = s_e[:, 0] + w_self0 + 1e-16
    s1 = s_e[:, 1] + w_self1 + 1e-16
    zz0 = (z[:, :C] + w_self0[:, None] * hh[:, :C]) / s0[:, None]
    zz1 = (z[:, C:] + w_self1[:, None] * hh[:, C:]) / s1[:, None]
    gat = jnp.concatenate([zz0, zz1], axis=1) + bg_ref[...][None, :]
    gat = jnp.maximum(gat, 0.0)
    t = jnp.maximum(jnp.dot(gat, wa_ref[...], preferred_element_type=jnp.float32) + ba_ref[...][None, :], 0.0)
    t = jnp.maximum(jnp.dot(t, w1_ref[...], preferred_element_type=jnp.float32) + b1_ref[...][None, :], 0.0)
    t = jnp.maximum(jnp.dot(t, w2_ref[...], preferred_element_type=jnp.float32) + b2_ref[...][None, :], 0.0)
    t = jnp.maximum(jnp.dot(t, w3_ref[...], preferred_element_type=jnp.float32) + b3_ref[...][None, :], 0.0)
    emb = jnp.dot(t, w4_ref[...], preferred_element_type=jnp.float32) + b4_ref[...][None, :]
    out_ref[...] = jnp.concatenate(
        [emb, jnp.zeros((emb.shape[0], 128 - emb.shape[1]), jnp.float32)], axis=1)


def _run_c(z, s, a_all, m_all, h_all, b_gat, Wa, ba, W1, b1, W2, b2, W3, b3, W4, b4):
    bm = 2000
    grid = (N // bm,)
    return pl.pallas_call(
        _c_kernel,
        grid=grid,
        in_specs=[
            pl.BlockSpec((bm, H * C), lambda i: (i, 0)),
            pl.BlockSpec((bm, 2), lambda i: (i, 0)),
            pl.BlockSpec((bm, 8), lambda i: (i, 0)),
            pl.BlockSpec((8, 128), lambda i: (0, 0)),
            pl.BlockSpec((bm, H * C), lambda i: (i, 0)),
            pl.BlockSpec((H * C,), lambda i: (0,)),
            pl.BlockSpec((512, 256), lambda i: (0, 0)),
            pl.BlockSpec((256,), lambda i: (0,)),
            pl.BlockSpec((256, 128), lambda i: (0, 0)),
            pl.BlockSpec((128,), lambda i: (0,)),
            pl.BlockSpec((128, 64), lambda i: (0, 0)),
            pl.BlockSpec((64,), lambda i: (0,)),
            pl.BlockSpec((64, 32), lambda i: (0, 0)),
            pl.BlockSpec((32,), lambda i: (0,)),
            pl.BlockSpec((32, 3), lambda i: (0, 0)),
            pl.BlockSpec((3,), lambda i: (0,)),
        ],
        out_specs=pl.BlockSpec((bm, 128), lambda i: (i, 0)),
        out_shape=jax.ShapeDtypeStruct((N, 128), jnp.float32),
    )(z, s, a_all, m_all, h_all, b_gat, Wa, ba, W1, b1, W2, b2, W3, b3, W4, b4)


def _d_kernel(ei_ref, ej_ref, out_ref):
    oi = ei_ref[...]
    oj = ej_ref[...]
    sqi = jnp.sum(oi * oi, axis=1)
    sqj = jnp.sum(oj * oj, axis=1)
    d2 = sqi[:, None] + sqj[None, :] - 2.0 * jnp.dot(
        oi, oj.T, preferred_element_type=jnp.float32)
    d2 = jnp.maximum(d2, 0.0)
    mask = d2 > 1e-12
    out_ref[...] = jnp.where(mask, jnp.sqrt(jnp.where(mask, d2, 1.0)), 0.0)


def _run_d(embp):
    bi = 400
    grid = (N // bi,)
    return pl.pallas_call(
        _d_kernel,
        grid=grid,
        in_specs=[
            pl.BlockSpec((bi, 128), lambda i: (i, 0)),
            pl.BlockSpec((N, 128), lambda i: (0, 0)),
        ],
        out_specs=pl.BlockSpec((bi, N), lambda i: (i, 0)),
        out_shape=jax.ShapeDtypeStruct((N, N), jnp.float32),
    )(embp, embp)


HALF = N // 2
ROUND = 1256
NR = 4
EPT = E // 16
EBLK = 2000
CAPQ = 1552
ZL = 1280


def _splat(v, k):
    return v[jnp.full((16,), k, jnp.int32)]


def _sc_body(src_hbm, dst_hbm, a4_hbm, m_hbm, h_hbm, z_hbm, s_hbm, sscr_hbm,
             a4_v, m_v, se_v, de_v, s_part,
             bsrc, bdst,
             rows0, rows1, sidx0, didx0, sidx1, didx1,
             row_tmp, acc_v, cnts_s,
             z_sp, sem0, sem0b, sem1):
    cid = lax.axis_index("c")
    sid = lax.axis_index("s")
    zeros16 = jnp.zeros((16,), jnp.float32)
    iota16 = lax.iota(jnp.int32, 16)

    pltpu.sync_copy(a4_hbm, a4_v)
    pltpu.sync_copy(m_hbm, m_v)
    mv = m_v[...]
    m0s = _splat(mv, 0)
    m1s = _splat(mv, 1)
    half_lo = cid * HALF

    def _zs(i, _):
        s_part[pl.ds(i * 16, 16)] = zeros16
        return 0
    lax.fori_loop(0, N // 16, _zs, 0)

    def _edge_w(srcv, dstv):
        as0 = plsc.load_gather(a4_v, [srcv])
        as1 = plsc.load_gather(a4_v, [N + srcv])
        ad0 = plsc.load_gather(a4_v, [2 * N + dstv])
        ad1 = plsc.load_gather(a4_v, [3 * N + dstv])
        e0 = as0 + ad0
        e1 = as1 + ad1
        e0 = jnp.where(e0 > 0, e0, NEG_SLOPE * e0)
        e1 = jnp.where(e1 > 0, e1, NEG_SLOPE * e1)
        return jnp.exp(e0 - m0s), jnp.exp(e1 - m1s)

    ebase = sid * EPT
    carry0 = (jnp.int32(0),) * NR

    def _blk(b, carry):
        pltpu.sync_copy(src_hbm.at[pl.ds(ebase + b * EBLK, EBLK)], se_v)
        pltpu.sync_copy(dst_hbm.at[pl.ds(ebase + b * EBLK, EBLK)], de_v)

        def _p1(g, pc):
            srcv = se_v[pl.ds(g * 16, 16)]
            dstv = de_v[pl.ds(g * 16, 16)]
            w0, w1 = _edge_w(srcv, dstv)
            dl = dstv - half_lo
            own = (dl >= 0) & (dl < HALF)
            plsc.addupdate_scatter(s_part, [2 * dl], w0, mask=own)
            plsc.addupdate_scatter(s_part, [2 * dl + 1], w1, mask=own)
            q = dl // ROUND
            dq = dl - q * ROUND
            newpc = []
            for rr in range(NR):
                mk = own & (q == rr)
                o = jnp.minimum(pc[rr], CAPQ - 16)
                plsc.store_compressed(bsrc.at[pl.ds(rr * CAPQ + o, 16)], srcv, mask=mk)
                plsc.store_compressed(bdst.at[pl.ds(rr * CAPQ + o, 16)], dq, mask=mk)
                newpc.append(pc[rr] + jnp.sum(mk.astype(jnp.int32)))
            return tuple(newpc)

        return lax.fori_loop(0, EBLK // 16, _p1, carry)

    cnts = lax.fori_loop(0, EPT // EBLK, _blk, carry0)
    for rr in range(NR):
        cnts_s[rr] = jnp.minimum(cnts[rr], CAPQ - 16)

    myrow = (cid * 16 + sid) * N
    pltpu.sync_copy(s_part, sscr_hbm.at[pl.ds(myrow, N)])
    plsc.subcore_barrier()
    rbase = jnp.minimum(sid * 632, N - 640)

    def _zacc(i, _):
        acc_v[pl.ds(i * 16, 16)] = zeros16
        return 0
    lax.fori_loop(0, 40, _zacc, 0)

    def _sred(j, _):
        pltpu.sync_copy(sscr_hbm.at[pl.ds((cid * 16 + j) * N + rbase, 640)], row_tmp)
        def _add(k, __):
            acc_v[pl.ds(k * 16, 16)] = acc_v[pl.ds(k * 16, 16)] + row_tmp[pl.ds(k * 16, 16)]
            return 0
        lax.fori_loop(0, 40, _add, 0)
        return 0
    lax.fori_loop(0, 16, _sred, 0)
    pltpu.sync_copy(acc_v, s_hbm.at[pl.ds(cid * N + rbase, 640)])

    ii4 = iota16 // 4
    im4 = iota16 % 4

    def _zero_rows(rv):
        for jj in range(16):
            for t in range(4):
                for u in range(8):
                    rv[jj, t, pl.ds(u * 16, 16)] = zeros16

    def _build(c, qbase, cnt, si, di):
        lanes = (iota16 + c * 16) < cnt
        rb = bb + c * 16
        srcv = jnp.where(lanes, bsrc[pl.ds(rb, 16)], 0)
        dqv = jnp.where(lanes, bdst[pl.ds(rb, 16)], 0)
        si[...] = srcv
        di[...] = dqv
        w0, w1 = _edge_w(srcv, qbase + dqv)
        w0 = jnp.where(lanes, w0, 0.0)
        w1 = jnp.where(lanes, w1, 0.0)
        return w0, w1

    def _scale(rv, w0, w1):
        for k in range(16):
            s0 = _splat(w0, k)
            s1 = _splat(w1, k)
            for t in range(4):
                f = s0 if t < 2 else s1
                for u in range(8):
                    rv[k, t, pl.ds(u * 16, 16)] = rv[k, t, pl.ds(u * 16, 16)] * f

    def _round(r, _):
        qbase = half_lo + r * ROUND
        _zero_rows(rows0)
        zb4 = sid * 80
        for kz in range(5):
            pltpu.sync_copy(rows0, z_sp.at[pl.ds(zb4 + kz * 16, 16)])
        plsc.subcore_barrier()

        cnt = cnts_s[r]
        nch2 = (cnt + 31) // 32
        global bb
        bb = r * CAPQ

        w00, w10 = _build(0, qbase, cnt, sidx0, didx0)
        pltpu.async_copy(h_hbm.at[sidx0], rows0, sem0)
        w01, w11 = _build(1, qbase, cnt, sidx1, didx1)
        pltpu.async_copy(h_hbm.at[sidx1], rows1, sem0b)

        def _p2(j2, carry):
            wa0, wa1, wb0, wb1 = carry
            ca = 2 * j2
            pltpu.make_async_copy(h_hbm.at[sidx0], rows0, sem0).wait()
            _scale(rows0, wa0, wa1)
            pltpu.async_copy(rows0, z_sp.at[didx0], sem1, add=True)
            pltpu.make_async_copy(h_hbm.at[sidx1], rows1, sem0b).wait()
            _scale(rows1, wb0, wb1)
            pltpu.async_copy(rows1, z_sp.at[didx1], sem1, add=True)
            pltpu.make_async_copy(rows0, z_sp.at[didx0], sem1).wait()
            na0, na1 = _build(ca + 2, qbase, cnt, sidx0, didx0)
            pltpu.async_copy(h_hbm.at[sidx0], rows0, sem0)
            pltpu.make_async_copy(rows1, z_sp.at[didx1], sem1).wait()
            nb0, nb1 = _build(ca + 3, qbase, cnt, sidx1, didx1)
            pltpu.async_copy(h_hbm.at[sidx1], rows1, sem0b)
            return na0, na1, nb0, nb1

        lax.fori_loop(0, nch2, _p2, (w00, w10, w01, w11))
        pltpu.make_async_copy(h_hbm.at[sidx0], rows0, sem0).wait()
        pltpu.make_async_copy(h_hbm.at[sidx1], rows1, sem0b).wait()
        plsc.subcore_barrier()
        zmax = jnp.where(r == NR - 1, 1152, ROUND - 80)
        zst = jnp.minimum(sid * 80, zmax)
        pltpu.sync_copy(z_sp.at[pl.ds(zst, 80)],
                        z_hbm.at[pl.ds(qbase + zst, 80)])
        plsc.subcore_barrier()
        return 0

    lax.fori_loop(0, NR, _round, 0)


def _run_b(src, dst, a4, m16, h_all):
    mesh = plsc.VectorSubcoreMesh(core_axis_name="c", subcore_axis_name="s")
    f32 = jnp.float32
    i32 = jnp.int32
    sc = pl.kernel(
        _sc_body,
        mesh=mesh,
        compiler_params=pltpu.CompilerParams(needs_layout_passes=False),
        out_type=[
            jax.ShapeDtypeStruct((N, 4, 128), f32),
            jax.ShapeDtypeStruct((2 * N,), f32),
            jax.ShapeDtypeStruct((32 * N,), f32),
        ],
        scratch_types=[
            pltpu.VMEM((4 * N,), f32),
            pltpu.VMEM((16,), f32),
            pltpu.VMEM((EBLK,), i32),
            pltpu.VMEM((EBLK,), i32),
            pltpu.VMEM((N,), f32),
            pltpu.VMEM((NR * CAPQ,), i32),
            pltpu.VMEM((NR * CAPQ,), i32),
            pltpu.VMEM((16, 4, 128), f32),
            pltpu.VMEM((16, 4, 128), f32),
            pltpu.VMEM((16,), i32),
            pltpu.VMEM((16,), i32),
            pltpu.VMEM((16,), i32),
            pltpu.VMEM((16,), i32),
            pltpu.VMEM((640,), f32),
            pltpu.VMEM((640,), f32),
            pltpu.SMEM((8,), i32),
            pltpu.VMEM_SHARED((ZL, 4, 128), f32),
            pltpu.SemaphoreType.DMA,
            pltpu.SemaphoreType.DMA,
            pltpu.SemaphoreType.DMA,
        ],
    )
    h4 = h_all.reshape((N, 4, 128))
    z4, s_flat, _sscr = sc(src, dst, a4.reshape(-1), m16, h4)
    z = z4.reshape(N, H * C)
    return z, s_flat.reshape(N, 2)


def kernel(x, edge_index, W, att_src, att_dst, b_gat, Wa, ba, W1, b1,
           W2, b2, W3, b3, W4, b4):
    src = edge_index[0]
    dst = edge_index[1]
    h_all, a_all, m_all = _run_a(x, W, att_src, att_dst)
    t0 = m_all[0, 0] + m_all[2, 0]
    t1 = m_all[1, 0] + m_all[3, 0]
    m0 = jnp.where(t0 > 0, t0, NEG_SLOPE * t0)
    m1 = jnp.where(t1 > 0, t1, NEG_SLOPE * t1)
    m2 = jnp.zeros((8, 128), jnp.float32)
    m2 = m2.at[0, :].set(m0).at[1, :].set(m1)
    a4 = jnp.transpose(a_all)[:4]
    m16 = jnp.zeros((16,), jnp.float32).at[0].set(m0).at[1].set(m1)
    z, s = _run_b(src, dst, a4, m16, h_all)
    embp = _run_c(z, s, a_all, m2, h_all, b_gat, Wa, ba,
                  W1, b1, W2, b2, W3, b3, W4, b4)
    return _run_d(embp)
